# baseline (speedup 1.0000x reference)
import functools

import jax
import jax.numpy as jnp
from jax import lax
from jax.experimental import pallas as pl
from jax.experimental.pallas import tpu as pltpu

N = 10000
E = 160000
H = 256
HEADS = 8
DH = 32
FFN = 1024
BN = 1000


def _ln(x, s, b):
    m = x.mean(-1, keepdims=True)
    v = ((x - m) ** 2).mean(-1, keepdims=True)
    return (x - m) / jnp.sqrt(v + 1e-5) * s + b


def _qkv_body(x_ref, ls_ref, lb_ref, wq_ref, bq_ref, wk_ref, bk_ref,
              wv_ref, bv_ref, q_ref, k_ref, v_ref):
    y = _ln(x_ref[...], ls_ref[...], lb_ref[...])
    q = jnp.dot(y, wq_ref[...], preferred_element_type=jnp.float32) + bq_ref[...]
    k = jnp.dot(y, wk_ref[...], preferred_element_type=jnp.float32) + bk_ref[...]
    v = jnp.dot(y, wv_ref[...], preferred_element_type=jnp.float32) + bv_ref[...]
    q_ref[0] = q[:, :128]
    q_ref[1] = q[:, 128:]
    k_ref[0] = k[:, :128]
    k_ref[1] = k[:, 128:]
    v_ref[0] = v[:, :128]
    v_ref[1] = v[:, 128:]


def _qkv_call(xf, ls, lb, wq, bq, wk, bk, wv, bv):
    grid = (N // BN,)
    row_spec = pl.BlockSpec((BN, H), lambda i: (i, 0))
    w_spec = pl.BlockSpec((H, H), lambda i: (0, 0))
    b_spec = pl.BlockSpec((1, H), lambda i: (0, 0))
    out_spec = pl.BlockSpec((2, BN, 128), lambda i: (0, i, 0))
    out_shape = jax.ShapeDtypeStruct((2, N, 128), jnp.float32)
    return pl.pallas_call(
        _qkv_body,
        grid=grid,
        in_specs=[row_spec, b_spec, b_spec, w_spec, b_spec, w_spec, b_spec,
                  w_spec, b_spec],
        out_specs=[out_spec, out_spec, out_spec],
        out_shape=[out_shape, out_shape, out_shape],
    )(xf, ls, lb, wq, bq, wk, bk, wv, bv)


def _tail_body(acc_ref, x_ref, wo_ref, bo_ref, ls_ref, lb_ref,
               w1_ref, b1_ref, w2_ref, b2_ref, out_ref):
    acc = acc_ref[...]
    msg = acc[:, :, :128]
    z = acc[:, :, 128:132]
    denom = jnp.reshape(
        jnp.broadcast_to(z[:, :, :, None], (2, BN, 4, DH)), (2, BN, 128))
    att2 = msg / (denom + 1e-6)
    att = jnp.concatenate([att2[0], att2[1]], axis=1)
    o = jnp.dot(att, wo_ref[...], preferred_element_type=jnp.float32) + bo_ref[...]
    x2 = x_ref[...] + o
    y2 = _ln(x2, ls_ref[...], lb_ref[...])
    h = jnp.dot(y2, w1_ref[...], preferred_element_type=jnp.float32) + b1_ref[...]
    h = 0.5 * h * (1.0 + lax.erf(h * 0.7071067811865476))
    h2 = jnp.dot(h, w2_ref[...], preferred_element_type=jnp.float32) + b2_ref[...]
    out_ref[...] = x2 + h2


def _tail_call(acc, xf, wo, bo, ls, lb, w1, b1, w2, b2):
    grid = (N // BN,)
    acc_spec = pl.BlockSpec((2, BN, 144), lambda i: (0, i, 0))
    row_spec = pl.BlockSpec((BN, H), lambda i: (i, 0))
    return pl.pallas_call(
        _tail_body,
        grid=grid,
        in_specs=[acc_spec, row_spec,
                  pl.BlockSpec((H, H), lambda i: (0, 0)),
                  pl.BlockSpec((1, H), lambda i: (0, 0)),
                  pl.BlockSpec((1, H), lambda i: (0, 0)),
                  pl.BlockSpec((1, H), lambda i: (0, 0)),
                  pl.BlockSpec((H, FFN), lambda i: (0, 0)),
                  pl.BlockSpec((1, FFN), lambda i: (0, 0)),
                  pl.BlockSpec((FFN, H), lambda i: (0, 0)),
                  pl.BlockSpec((1, H), lambda i: (0, 0))],
        out_specs=pl.BlockSpec((BN, H), lambda i: (i, 0)),
        out_shape=jax.ShapeDtypeStruct((N, H), jnp.float32),
    )(acc, xf, wo, bo, ls, lb, w1, b1, w2, b2)


def _edge_phase_jnp(q2, k2, v2, edge_index):
    q = jnp.concatenate([q2[0], q2[1]], axis=1).reshape(N, HEADS, DH)
    k = jnp.concatenate([k2[0], k2[1]], axis=1).reshape(N, HEADS, DH)
    v = jnp.concatenate([v2[0], v2[1]], axis=1).reshape(N, HEADS, DH)
    scale = jnp.sqrt(jnp.float32(DH))
    src = jnp.take(k, edge_index[0], axis=0)
    dst = jnp.take(q, edge_index[1], axis=0)
    score = jnp.exp(jnp.clip((src * dst).sum(-1, keepdims=True) / scale, -5.0, 5.0))
    msg = jnp.take(v, edge_index[0], axis=0) * score
    wV = jnp.zeros_like(v).at[edge_index[1]].add(msg)
    Z = jnp.zeros((N, HEADS, 1), v.dtype).at[edge_index[1]].add(score)
    wv_flat = wV.reshape(N, H)
    z_flat = Z.reshape(N, HEADS)
    acc = jnp.zeros((2, N, 144), jnp.float32)
    acc = acc.at[0, :, :128].set(wv_flat[:, :128])
    acc = acc.at[1, :, :128].set(wv_flat[:, 128:])
    acc = acc.at[0, :, 128:132].set(z_flat[:, :4])
    acc = acc.at[1, :, 128:132].set(z_flat[:, 4:])
    return acc


def kernel(x, edge_index, ln1_scale, ln1_bias, Wq, bq, Wk, bk, Wv, bv,
           Wo, bo, ln2_scale, ln2_bias, W1, b1, W2, b2):
    xf = x.reshape(N, H)
    r = lambda a: a.reshape(1, -1)
    q2, k2, v2 = _qkv_call(xf, r(ln1_scale), r(ln1_bias), Wq, r(bq),
                           Wk, r(bk), Wv, r(bv))
    acc = _edge_phase_jnp(q2, k2, v2, edge_index)
    out = _tail_call(acc, xf, Wo, r(bo), r(ln2_scale), r(ln2_bias),
                     W1, r(b1), W2, r(b2))
    return out.reshape(1, N, H)


# speedup vs baseline: 16.4840x; 16.4840x over previous
import functools

import jax
import jax.numpy as jnp
from jax import lax
from jax.experimental import pallas as pl
from jax.experimental.pallas import tpu as pltpu
from jax.experimental.pallas import tpu_sc as plsc

N = 10000
E = 160000
H = 256
HEADS = 8
DH = 32
FFN = 1024
BN = 1000
BT = 2048
BE = 2000
NPAD = 10240
NZ = NPAD // 16
_INV_SCALE = 1.0 / (DH ** 0.5)


def _ln(x, s, b):
    m = x.mean(-1, keepdims=True)
    v = ((x - m) ** 2).mean(-1, keepdims=True)
    return (x - m) / jnp.sqrt(v + 1e-5) * s + b


def _qkv_body(x_ref, ls_ref, lb_ref, wq_ref, bq_ref, wk_ref, bk_ref,
              wv_ref, bv_ref, q_ref, kv_ref):
    y = _ln(x_ref[...], ls_ref[...], lb_ref[...])
    q = jnp.dot(y, wq_ref[...], preferred_element_type=jnp.float32) + bq_ref[...]
    k = jnp.dot(y, wk_ref[...], preferred_element_type=jnp.float32) + bk_ref[...]
    v = jnp.dot(y, wv_ref[...], preferred_element_type=jnp.float32) + bv_ref[...]
    q_ref[...] = q
    kv_ref[:, :H] = k
    kv_ref[:, H:] = v


def _qkv_call(xf, ls, lb, wq, bq, wk, bk, wv, bv):
    grid = (N // BN,)
    row_spec = pl.BlockSpec((BN, H), lambda i: (i, 0))
    w_spec = pl.BlockSpec((H, H), lambda i: (0, 0))
    b_spec = pl.BlockSpec((1, H), lambda i: (0, 0))
    return pl.pallas_call(
        _qkv_body,
        grid=grid,
        in_specs=[row_spec, b_spec, b_spec, w_spec, b_spec, w_spec, b_spec,
                  w_spec, b_spec],
        out_specs=[pl.BlockSpec((BN, H), lambda i: (i, 0)),
                   pl.BlockSpec((BN, 2 * H), lambda i: (i, 0))],
        out_shape=[jax.ShapeDtypeStruct((N, H), jnp.float32),
                   jax.ShapeDtypeStruct((N, 2 * H), jnp.float32)],
    )(xf, ls, lb, wq, bq, wk, bk, wv, bv)


GCH = 40
GEPW = E // 32


def _gather_sc_kernel(kv_ref, q_ref, src_ref, dst_ref, kvg_ref, qg_ref,
                      ibuf, jbuf, kvbuf, qbuf, sem):
    c = lax.axis_index("c")
    s = lax.axis_index("s")
    w = s * 2 + c

    def _chunk(i, _):
        base = w * GEPW + i * GCH
        pltpu.sync_copy(src_ref.at[pl.ds(base, GCH)], ibuf)
        pltpu.sync_copy(dst_ref.at[pl.ds(base, GCH)], jbuf)
        cp1 = pltpu.async_copy(kv_ref.at[ibuf], kvbuf, sem)
        cp2 = pltpu.async_copy(q_ref.at[jbuf], qbuf, sem)
        cp1.wait()
        cp2.wait()
        pltpu.sync_copy(kvbuf, kvg_ref.at[pl.ds(base, GCH)])
        pltpu.sync_copy(qbuf, qg_ref.at[pl.ds(base, GCH)])
        return _
    lax.fori_loop(0, GEPW // GCH, _chunk, None)


def _gather_sc(kv, q, src, dst):
    mesh = plsc.VectorSubcoreMesh(core_axis_name="c", subcore_axis_name="s")
    f = functools.partial(
        pl.kernel,
        mesh=mesh,
        out_type=[jax.ShapeDtypeStruct((E, 2 * H), jnp.float32),
                  jax.ShapeDtypeStruct((E, H), jnp.float32)],
        scratch_types=[
            pltpu.VMEM((GCH,), jnp.int32),
            pltpu.VMEM((GCH,), jnp.int32),
            pltpu.VMEM((GCH, 2 * H), jnp.float32),
            pltpu.VMEM((GCH, H), jnp.float32),
            pltpu.SemaphoreType.DMA,
        ],
    )(_gather_sc_kernel)
    return f(kv, q, src, dst)


def _score_body(kvg_ref, qg_ref, dst_ref, mask_ref, msg_ref, z_ref):
    kvg = kvg_ref[...]
    k = kvg[:, :H]
    v = kvg[:, H:]
    qg = qg_ref[...]
    kq = k * qg
    s = jnp.dot(kq, mask_ref[...], preferred_element_type=jnp.float32)
    score = jnp.exp(jnp.clip(s * _INV_SCALE, -5.0, 5.0))
    hrow = lax.broadcasted_iota(jnp.int32, (HEADS, H), 0)
    hcol = lax.broadcasted_iota(jnp.int32, (HEADS, H), 1) // DH
    expand = (hrow == hcol).astype(jnp.float32)
    sfull = jnp.dot(score, expand, preferred_element_type=jnp.float32)
    msg = v * sfull
    msg_ref[0] = msg[:, :128]
    msg_ref[1] = msg[:, 128:]
    dstv = dst_ref[0, 0, :]
    trow = lax.broadcasted_iota(jnp.int32, (HEADS, 128), 0)
    tcol = lax.broadcasted_iota(jnp.int32, (HEADS, 128), 1) % HEADS
    tile16 = (trow == tcol).astype(jnp.float32)
    scoretile = jnp.dot(score, tile16, preferred_element_type=jnp.float32)
    lane = lax.broadcasted_iota(jnp.int32, (BE, 128), 1) // HEADS
    slot = jnp.broadcast_to((dstv % 16)[:, None], (BE, 128))
    z_ref[...] = jnp.where(lane == slot, scoretile, 0.0)


def _score_call(kvg, qg, dstb):
    grid = (E // BE,)
    mask = jnp.reshape(
        jnp.broadcast_to(jnp.eye(HEADS, dtype=jnp.float32)[:, None, :],
                         (HEADS, DH, HEADS)), (H, HEADS))
    return pl.pallas_call(
        _score_body,
        grid=grid,
        in_specs=[pl.BlockSpec((BE, 2 * H), lambda i: (i, 0)),
                  pl.BlockSpec((BE, H), lambda i: (i, 0)),
                  pl.BlockSpec((1, 1, BE), lambda i: (i, 0, 0)),
                  pl.BlockSpec((H, HEADS), lambda i: (0, 0))],
        out_specs=[pl.BlockSpec((2, BE, 128), lambda i: (0, i, 0)),
                   pl.BlockSpec((BE, 128), lambda i: (i, 0))],
        out_shape=[jax.ShapeDtypeStruct((2, E, 128), jnp.float32),
                   jax.ShapeDtypeStruct((E, 128), jnp.float32)],
    )(kvg, qg, dstb, mask)


SCH = 80
SEPW = E // 16
ROWS_PT = NPAD // 16
ZROWS_PT = NZ // 16


def _scatter_sc_kernel(msg_ref, zrows_ref, dst_ref, dstdiv_ref,
                       outm_ref, outz_ref,
                       jbuf, j2buf, mbuf, zsrc, zstage, accm_sh, accz_sh,
                       sem):
    c = lax.axis_index("c")
    s = lax.axis_index("s")
    zeros16 = jnp.zeros((16,), jnp.float32)

    def _zrow(r, _):
        for j in range(128 // 16):
            zstage[r, pl.ds(j * 16, 16)] = zeros16
        return _
    lax.fori_loop(0, 128, _zrow, None)

    def _zcp(t, _):
        pltpu.sync_copy(zstage, accm_sh.at[pl.ds(s * ROWS_PT + t * 128, 128)])
        return _
    lax.fori_loop(0, ROWS_PT // 128, _zcp, None)
    pltpu.sync_copy(zstage.at[pl.ds(0, ZROWS_PT)],
                    accz_sh.at[pl.ds(s * ZROWS_PT, ZROWS_PT)])

    plsc.subcore_barrier()

    def _chunk(i, _):
        base = s * SEPW + i * SCH
        pltpu.sync_copy(dst_ref.at[pl.ds(base, SCH)], jbuf)
        pltpu.sync_copy(dstdiv_ref.at[pl.ds(base, SCH)], j2buf)
        pltpu.sync_copy(msg_ref.at[c, pl.ds(base, SCH)], mbuf)
        pltpu.sync_copy(zrows_ref.at[pl.ds(base, SCH)], zsrc)
        pltpu.sync_copy(mbuf, accm_sh.at[jbuf], add=True)
        pltpu.sync_copy(zsrc, accz_sh.at[j2buf], add=True)
        return _
    lax.fori_loop(0, SEPW // SCH, _chunk, None)

    plsc.subcore_barrier()
    pltpu.sync_copy(accm_sh.at[pl.ds(s * ROWS_PT, ROWS_PT)],
                    outm_ref.at[c, pl.ds(s * ROWS_PT, ROWS_PT)])
    pltpu.sync_copy(accz_sh.at[pl.ds(s * ZROWS_PT, ZROWS_PT)],
                    outz_ref.at[c, pl.ds(s * ZROWS_PT, ZROWS_PT)])


def _scatter_sc(msg, zrows, dst, dstdiv):
    mesh = plsc.VectorSubcoreMesh(core_axis_name="c", subcore_axis_name="s")
    f = functools.partial(
        pl.kernel,
        mesh=mesh,
        out_type=[jax.ShapeDtypeStruct((2, NPAD, 128), jnp.float32),
                  jax.ShapeDtypeStruct((2, NZ, 128), jnp.float32)],
        scratch_types=[
            pltpu.VMEM((SCH,), jnp.int32),
            pltpu.VMEM((SCH,), jnp.int32),
            pltpu.VMEM((SCH, 128), jnp.float32),
            pltpu.VMEM((SCH, 128), jnp.float32),
            pltpu.VMEM((128, 128), jnp.float32),
            pltpu.VMEM_SHARED((NPAD, 128), jnp.float32),
            pltpu.VMEM_SHARED((NZ, 128), jnp.float32),
            pltpu.SemaphoreType.DMA,
        ],
    )(_scatter_sc_kernel)
    return f(msg, zrows, dst, dstdiv)


def _tail_body(acc_ref, zacc_ref, x_ref, wo_ref, bo_ref, ls_ref, lb_ref,
               w1_ref, b1_ref, w2_ref, b2_ref, out_ref):
    acc = acc_ref[...]
    zr = zacc_ref[...]
    parts = []
    for g in range(2):
        prow = lax.broadcasted_iota(jnp.int32, (HEADS, 128), 0)
        pcol = lax.broadcasted_iota(jnp.int32, (HEADS, 128), 1) // DH + 4 * g
        sel = (prow == pcol).astype(jnp.float32)
        denom = jnp.dot(zr[g], sel, preferred_element_type=jnp.float32)
        parts.append(acc[g] / (denom + 1e-6))
    att = jnp.concatenate(parts, axis=1)
    o = jnp.dot(att, wo_ref[...], preferred_element_type=jnp.float32) + bo_ref[...]
    x2 = x_ref[...] + o
    y2 = _ln(x2, ls_ref[...], lb_ref[...])
    h = jnp.dot(y2, w1_ref[...], preferred_element_type=jnp.float32) + b1_ref[...]
    h = 0.5 * h * (1.0 + lax.erf(h * 0.7071067811865476))
    h2 = jnp.dot(h, w2_ref[...], preferred_element_type=jnp.float32) + b2_ref[...]
    out_ref[...] = x2 + h2


def _tail_call(acc, zacc, xf, wo, bo, ls, lb, w1, b1, w2, b2):
    grid = (NPAD // BT,)
    return pl.pallas_call(
        _tail_body,
        grid=grid,
        in_specs=[pl.BlockSpec((2, BT, 128), lambda i: (0, i, 0)),
                  pl.BlockSpec((2, BT, 8), lambda i: (0, i, 0)),
                  pl.BlockSpec((BT, H), lambda i: (i, 0)),
                  pl.BlockSpec((H, H), lambda i: (0, 0)),
                  pl.BlockSpec((1, H), lambda i: (0, 0)),
                  pl.BlockSpec((1, H), lambda i: (0, 0)),
                  pl.BlockSpec((1, H), lambda i: (0, 0)),
                  pl.BlockSpec((H, FFN), lambda i: (0, 0)),
                  pl.BlockSpec((1, FFN), lambda i: (0, 0)),
                  pl.BlockSpec((FFN, H), lambda i: (0, 0)),
                  pl.BlockSpec((1, H), lambda i: (0, 0))],
        out_specs=pl.BlockSpec((BT, H), lambda i: (i, 0)),
        out_shape=jax.ShapeDtypeStruct((N, H), jnp.float32),
    )(acc, zacc, xf, wo, bo, ls, lb, w1, b1, w2, b2)


def kernel(x, edge_index, ln1_scale, ln1_bias, Wq, bq, Wk, bk, Wv, bv,
           Wo, bo, ln2_scale, ln2_bias, W1, b1, W2, b2):
    xf = x.reshape(N, H)
    r = lambda a: a.reshape(1, -1)
    q, kv = _qkv_call(xf, r(ln1_scale), r(ln1_bias), Wq, r(bq),
                      Wk, r(bk), Wv, r(bv))
    src = edge_index[0]
    dst = edge_index[1]
    dstdiv = jnp.right_shift(dst, 4)
    dstb = dst.reshape(E // BE, 1, BE)
    kvg, qg = _gather_sc(kv, q, src, dst)
    msg, zrows = _score_call(kvg, qg, dstb)
    accm, accz = _scatter_sc(msg, zrows, dst, dstdiv)
    accz = accz.reshape(2, NPAD, HEADS)
    out = _tail_call(accm, accz, xf, Wo, r(bo), r(ln2_scale), r(ln2_bias),
                     W1, r(b1), W2, r(b2))
    return out.reshape(1, N, H)


# speedup vs baseline: 18.6054x; 1.1287x over previous
import functools

import jax
import jax.numpy as jnp
from jax import lax
from jax.experimental import pallas as pl
from jax.experimental.pallas import tpu as pltpu
from jax.experimental.pallas import tpu_sc as plsc

N = 10000
E = 160000
H = 256
HEADS = 8
DH = 32
FFN = 1024
BN = 1000
BT = 2048
BE = 2000
NPAD = 10240
NZ = NPAD // 16
_INV_SCALE = 1.0 / (DH ** 0.5)


def _ln(x, s, b):
    m = x.mean(-1, keepdims=True)
    v = ((x - m) ** 2).mean(-1, keepdims=True)
    return (x - m) / jnp.sqrt(v + 1e-5) * s + b


def _qkv_body(x_ref, ls_ref, lb_ref, wq_ref, bq_ref, wk_ref, bk_ref,
              wv_ref, bv_ref, q_ref, kv_ref):
    y = _ln(x_ref[...], ls_ref[...], lb_ref[...])
    q = jnp.dot(y, wq_ref[...], preferred_element_type=jnp.float32) + bq_ref[...]
    k = jnp.dot(y, wk_ref[...], preferred_element_type=jnp.float32) + bk_ref[...]
    v = jnp.dot(y, wv_ref[...], preferred_element_type=jnp.float32) + bv_ref[...]
    def bits(a):
        rounded = a.astype(jnp.bfloat16).astype(jnp.float32)
        return lax.bitcast_convert_type(rounded, jnp.int32)

    himask = jnp.int32(-65536)
    kw = lax.shift_right_logical(bits(k), 16)
    vw = jnp.bitwise_and(bits(v), himask)
    kv_ref[...] = jnp.bitwise_or(kw, vw)
    qlo = lax.shift_right_logical(bits(q[:, :128]), 16)
    qhi = jnp.bitwise_and(bits(q[:, 128:]), himask)
    q_ref[...] = jnp.bitwise_or(qlo, qhi)


def _qkv_call(xf, ls, lb, wq, bq, wk, bk, wv, bv):
    grid = (N // BN,)
    row_spec = pl.BlockSpec((BN, H), lambda i: (i, 0))
    w_spec = pl.BlockSpec((H, H), lambda i: (0, 0))
    b_spec = pl.BlockSpec((1, H), lambda i: (0, 0))
    return pl.pallas_call(
        _qkv_body,
        grid=grid,
        in_specs=[row_spec, b_spec, b_spec, w_spec, b_spec, w_spec, b_spec,
                  w_spec, b_spec],
        out_specs=[pl.BlockSpec((BN, 128), lambda i: (i, 0)),
                   pl.BlockSpec((BN, H), lambda i: (i, 0))],
        out_shape=[jax.ShapeDtypeStruct((N, 128), jnp.int32),
                   jax.ShapeDtypeStruct((N, H), jnp.int32)],
    )(xf, ls, lb, wq, bq, wk, bk, wv, bv)


GCH = 40
GEPW = E // 32
GNCH = GEPW // GCH


def _gather_sc_kernel(kv_ref, q_ref, src_ref, dst_ref, kvg_ref, qg_ref,
                      ibuf0, jbuf0, kvb0, qb0,
                      ibuf1, jbuf1, kvb1, qb1, sem0, sem1):
    c = lax.axis_index("c")
    s = lax.axis_index("s")
    w = s * 2 + c
    ib = (ibuf0, ibuf1)
    jb = (jbuf0, jbuf1)
    kvb = (kvb0, kvb1)
    qb = (qb0, qb1)
    sems = (sem0, sem1)

    def start(ci):
        b = ci % 2
        base = w * GEPW + ci * GCH
        pltpu.sync_copy(src_ref.at[pl.ds(base, GCH)], ib[b])
        pltpu.sync_copy(dst_ref.at[pl.ds(base, GCH)], jb[b])
        return (pltpu.async_copy(kv_ref.at[ib[b]], kvb[b], sems[b]),
                pltpu.async_copy(q_ref.at[jb[b]], qb[b], sems[b]))

    def finish(ci, cps):
        b = ci % 2
        for cp in cps:
            cp.wait()
        base = w * GEPW + ci * GCH
        pltpu.sync_copy(kvb[b], kvg_ref.at[pl.ds(base, GCH)])
        pltpu.sync_copy(qb[b], qg_ref.at[pl.ds(base, GCH)])

    pending = {0: start(0)}
    for ci in range(GNCH):
        if ci + 1 < GNCH:
            pending[ci + 1] = start(ci + 1)
        finish(ci, pending.pop(ci))


def _gather_sc(kv, q, src, dst):
    mesh = plsc.VectorSubcoreMesh(core_axis_name="c", subcore_axis_name="s")
    f = functools.partial(
        pl.kernel,
        mesh=mesh,
        out_type=[jax.ShapeDtypeStruct((E, H), jnp.int32),
                  jax.ShapeDtypeStruct((E, 128), jnp.int32)],
        scratch_types=[
            pltpu.VMEM((GCH,), jnp.int32),
            pltpu.VMEM((GCH,), jnp.int32),
            pltpu.VMEM((GCH, H), jnp.int32),
            pltpu.VMEM((GCH, 128), jnp.int32),
            pltpu.VMEM((GCH,), jnp.int32),
            pltpu.VMEM((GCH,), jnp.int32),
            pltpu.VMEM((GCH, H), jnp.int32),
            pltpu.VMEM((GCH, 128), jnp.int32),
            pltpu.SemaphoreType.DMA,
            pltpu.SemaphoreType.DMA,
        ],
    )(_gather_sc_kernel)
    return f(kv, q, src, dst)


def _head_selector(width, offset):
    hrow = lax.broadcasted_iota(jnp.int32, (HEADS, width), 0)
    hcol = (lax.broadcasted_iota(jnp.int32, (HEADS, width), 1) + offset) // DH
    return (hrow == hcol).astype(jnp.float32)


def _score_body(kvg_ref, qg_ref, dst_ref, msg_ref, z_ref):
    himask = jnp.int32(-65536)
    lo = lambda w: lax.bitcast_convert_type(lax.shift_left(w, 16), jnp.float32)
    hi = lambda w: lax.bitcast_convert_type(jnp.bitwise_and(w, himask),
                                            jnp.float32)
    qw = qg_ref[...]
    qs = (lo(qw), hi(qw))
    kvw = (kvg_ref[:, :128], kvg_ref[:, 128:])
    score_parts = []
    for g in range(2):
        kq = lo(kvw[g]) * qs[g]
        sel = _head_selector(128, g * 128)
        score_parts.append(
            jnp.dot(kq, sel.T, preferred_element_type=jnp.float32))
    s = score_parts[0] + score_parts[1]
    score = jnp.exp(jnp.clip(s * _INV_SCALE, -5.0, 5.0))
    for g in range(2):
        sfull = jnp.dot(score, _head_selector(128, g * 128),
                        preferred_element_type=jnp.float32)
        msg_ref[g] = hi(kvw[g]) * sfull
    dstv = dst_ref[0, 0, :]
    trow = lax.broadcasted_iota(jnp.int32, (HEADS, 128), 0)
    tcol = lax.broadcasted_iota(jnp.int32, (HEADS, 128), 1) % HEADS
    tile16 = (trow == tcol).astype(jnp.float32)
    scoretile = jnp.dot(score, tile16, preferred_element_type=jnp.float32)
    lane = lax.broadcasted_iota(jnp.int32, (BE, 128), 1) // HEADS
    slot = jnp.broadcast_to((dstv % 16)[:, None], (BE, 128))
    z_ref[...] = jnp.where(lane == slot, scoretile, 0.0)


def _score_call(kvg, qg, dstb):
    grid = (E // BE,)
    return pl.pallas_call(
        _score_body,
        grid=grid,
        in_specs=[pl.BlockSpec((BE, H), lambda i: (i, 0)),
                  pl.BlockSpec((BE, 128), lambda i: (i, 0)),
                  pl.BlockSpec((1, 1, BE), lambda i: (i, 0, 0))],
        out_specs=[pl.BlockSpec((2, BE, 128), lambda i: (0, i, 0)),
                   pl.BlockSpec((BE, 128), lambda i: (i, 0))],
        out_shape=[jax.ShapeDtypeStruct((2, E, 128), jnp.float32),
                   jax.ShapeDtypeStruct((E, 128), jnp.float32)],
    )(kvg, qg, dstb)


SCH = 40
SEPW = E // 16
SNCH = SEPW // SCH
ROWS_PT = NPAD // 16
ZROWS_PT = NZ // 16


def _scatter_sc_kernel(msg_ref, zrows_ref, dst_ref, dstdiv_ref,
                       outm_ref, outz_ref,
                       jbuf0, j2buf0, mbuf0, zsrc0,
                       jbuf1, j2buf1, mbuf1, zsrc1,
                       zstage, accm_sh, accz_sh, sem0, sem1):
    c = lax.axis_index("c")
    s = lax.axis_index("s")
    zeros16 = jnp.zeros((16,), jnp.float32)
    jb = (jbuf0, jbuf1)
    j2b = (j2buf0, j2buf1)
    mb = (mbuf0, mbuf1)
    zb = (zsrc0, zsrc1)
    sems = (sem0, sem1)

    def _zrow(r, _):
        for j in range(128 // 16):
            zstage[r, pl.ds(j * 16, 16)] = zeros16
        return _
    lax.fori_loop(0, 64, _zrow, None)

    def _zcp(t, _):
        pltpu.sync_copy(zstage, accm_sh.at[pl.ds(s * ROWS_PT + t * 64, 64)])
        return _
    lax.fori_loop(0, ROWS_PT // 64, _zcp, None)
    pltpu.sync_copy(zstage.at[pl.ds(0, ZROWS_PT)],
                    accz_sh.at[pl.ds(s * ZROWS_PT, ZROWS_PT)])

    plsc.subcore_barrier()

    def _load(ci, b):
        base = s * SEPW + ci * SCH
        pltpu.sync_copy(dst_ref.at[pl.ds(base, SCH)], jb[b])
        pltpu.sync_copy(dstdiv_ref.at[pl.ds(base, SCH)], j2b[b])
        pltpu.sync_copy(msg_ref.at[c, pl.ds(base, SCH)], mb[b])
        pltpu.sync_copy(zrows_ref.at[pl.ds(base, SCH)], zb[b])

    def _fire(b):
        return (pltpu.async_copy(mb[b], accm_sh.at[jb[b]], sems[b], add=True),
                pltpu.async_copy(zb[b], accz_sh.at[j2b[b]], sems[b], add=True))

    def _body(i, _):
        ci0 = 2 * i
        _load(ci0, 0)
        cps0 = _fire(0)
        _load(ci0 + 1, 1)
        for cp in cps0:
            cp.wait()
        cps1 = _fire(1)
        for cp in cps1:
            cp.wait()
        return _
    lax.fori_loop(0, SNCH // 2, _body, None)

    plsc.subcore_barrier()
    pltpu.sync_copy(accm_sh.at[pl.ds(s * ROWS_PT, ROWS_PT)],
                    outm_ref.at[c, pl.ds(s * ROWS_PT, ROWS_PT)])
    pltpu.sync_copy(accz_sh.at[pl.ds(s * ZROWS_PT, ZROWS_PT)],
                    outz_ref.at[c, pl.ds(s * ZROWS_PT, ZROWS_PT)])


def _scatter_sc(msg, zrows, dst, dstdiv):
    mesh = plsc.VectorSubcoreMesh(core_axis_name="c", subcore_axis_name="s")
    f = functools.partial(
        pl.kernel,
        mesh=mesh,
        out_type=[jax.ShapeDtypeStruct((2, NPAD, 128), jnp.float32),
                  jax.ShapeDtypeStruct((2, NZ, 128), jnp.float32)],
        scratch_types=[
            pltpu.VMEM((SCH,), jnp.int32),
            pltpu.VMEM((SCH,), jnp.int32),
            pltpu.VMEM((SCH, 128), jnp.float32),
            pltpu.VMEM((SCH, 128), jnp.float32),
            pltpu.VMEM((SCH,), jnp.int32),
            pltpu.VMEM((SCH,), jnp.int32),
            pltpu.VMEM((SCH, 128), jnp.float32),
            pltpu.VMEM((SCH, 128), jnp.float32),
            pltpu.VMEM((64, 128), jnp.float32),
            pltpu.VMEM_SHARED((NPAD, 128), jnp.float32),
            pltpu.VMEM_SHARED((NZ, 128), jnp.float32),
            pltpu.SemaphoreType.DMA,
            pltpu.SemaphoreType.DMA,
        ],
    )(_scatter_sc_kernel)
    return f(msg, zrows, dst, dstdiv)


def _tail_body(acc_ref, zacc_ref, x_ref, wo_ref, bo_ref, ls_ref, lb_ref,
               w1_ref, b1_ref, w2_ref, b2_ref, out_ref):
    acc = acc_ref[...]
    zr = zacc_ref[...]
    parts = []
    for g in range(2):
        sel = _head_selector(128, g * 128)
        denom = jnp.dot(zr[g], sel, preferred_element_type=jnp.float32)
        parts.append(acc[g] / (denom + 1e-6))
    att = jnp.concatenate(parts, axis=1)
    o = jnp.dot(att, wo_ref[...], preferred_element_type=jnp.float32) + bo_ref[...]
    x2 = x_ref[...] + o
    y2 = _ln(x2, ls_ref[...], lb_ref[...])
    h = jnp.dot(y2, w1_ref[...], preferred_element_type=jnp.float32) + b1_ref[...]
    h = 0.5 * h * (1.0 + lax.erf(h * 0.7071067811865476))
    h2 = jnp.dot(h, w2_ref[...], preferred_element_type=jnp.float32) + b2_ref[...]
    out_ref[...] = x2 + h2


def _tail_call(acc, zacc, xf, wo, bo, ls, lb, w1, b1, w2, b2):
    grid = (NPAD // BT,)
    return pl.pallas_call(
        _tail_body,
        grid=grid,
        in_specs=[pl.BlockSpec((2, BT, 128), lambda i: (0, i, 0)),
                  pl.BlockSpec((2, BT, 8), lambda i: (0, i, 0)),
                  pl.BlockSpec((BT, H), lambda i: (i, 0)),
                  pl.BlockSpec((H, H), lambda i: (0, 0)),
                  pl.BlockSpec((1, H), lambda i: (0, 0)),
                  pl.BlockSpec((1, H), lambda i: (0, 0)),
                  pl.BlockSpec((1, H), lambda i: (0, 0)),
                  pl.BlockSpec((H, FFN), lambda i: (0, 0)),
                  pl.BlockSpec((1, FFN), lambda i: (0, 0)),
                  pl.BlockSpec((FFN, H), lambda i: (0, 0)),
                  pl.BlockSpec((1, H), lambda i: (0, 0))],
        out_specs=pl.BlockSpec((BT, H), lambda i: (i, 0)),
        out_shape=jax.ShapeDtypeStruct((N, H), jnp.float32),
    )(acc, zacc, xf, wo, bo, ls, lb, w1, b1, w2, b2)


def kernel(x, edge_index, ln1_scale, ln1_bias, Wq, bq, Wk, bk, Wv, bv,
           Wo, bo, ln2_scale, ln2_bias, W1, b1, W2, b2):
    xf = x.reshape(N, H)
    r = lambda a: a.reshape(1, -1)
    q, kv = _qkv_call(xf, r(ln1_scale), r(ln1_bias), Wq, r(bq),
                      Wk, r(bk), Wv, r(bv))
    src = edge_index[0]
    dst = edge_index[1]
    dstdiv = jnp.right_shift(dst, 4)
    dstb = dst.reshape(E // BE, 1, BE)
    kvg, qg = _gather_sc(kv, q, src, dst)
    msg, zrows = _score_call(kvg, qg, dstb)
    accm, accz = _scatter_sc(msg, zrows, dst, dstdiv)
    accz = accz.reshape(2, NPAD, HEADS)
    out = _tail_call(accm, accz, xf, Wo, r(bo), r(ln2_scale), r(ln2_bias),
                     W1, r(b1), W2, r(b2))
    return out.reshape(1, N, H)
